# baseline (speedup 1.0000x reference)
import functools

import jax
import jax.numpy as jnp
from jax import lax
from jax.experimental import pallas as pl
from jax.experimental.pallas import tpu as pltpu
from jax.experimental.pallas import tpu_sc as plsc

EMB = 1024
LANES = 16
VPR = EMB // LANES

_info = plsc.get_sparse_core_info()
NC, NS = _info.num_cores, _info.num_subcores
NW = NC * NS

NIN = 4
NOUT = 2


def _make_kernel(n_rows: int, c_rows: int):
    rows_per_w = n_rows // NW
    n_chunks = rows_per_w // c_rows
    assert (n_chunks - 4) % NIN == 0 and n_chunks >= 8
    mesh = plsc.VectorSubcoreMesh(core_axis_name="c", subcore_axis_name="s")

    buf = lambda: pltpu.VMEM((c_rows, EMB), jnp.float32)
    sem = pltpu.SemaphoreType.DMA

    @functools.partial(
        pl.kernel,
        mesh=mesh,
        out_type=jax.ShapeDtypeStruct((n_rows, EMB), jnp.float32),
        scratch_types=[
            pltpu.VMEM((rows_per_w,), jnp.int32),
            [buf() for _ in range(NIN)],
            [buf() for _ in range(NIN)],
            [buf() for _ in range(NOUT)],
            [sem for _ in range(NIN)],
            [sem for _ in range(NIN)],
            [sem for _ in range(NOUT)],
        ],
    )
    def k(emb_hbm, ts_hbm, table_hbm, out_hbm, idx_v, embs, rows, outs, ses, sgs, sos):
        wid = lax.axis_index("s") * NC + lax.axis_index("c")
        base = wid * rows_per_w
        pltpu.sync_copy(ts_hbm.at[pl.ds(base, rows_per_w)], idx_v)

        def start_in(ci, b):
            pltpu.async_copy(
                table_hbm.at[idx_v.at[pl.ds(ci * c_rows, c_rows)]], rows[b], sgs[b])
            pltpu.async_copy(
                emb_hbm.at[pl.ds(base + ci * c_rows, c_rows)], embs[b], ses[b])

        def wait_in(b):
            pltpu.make_async_copy(
                table_hbm.at[idx_v.at[pl.ds(0, c_rows)]], rows[b], sgs[b]).wait()
            pltpu.make_async_copy(
                emb_hbm.at[pl.ds(base, c_rows)], embs[b], ses[b]).wait()

        def add(bi, bo):
            @pl.loop(0, c_rows)
            def _(r):
                for j in range(VPR):
                    sl = pl.ds(j * LANES, LANES)
                    outs[bo][r, sl] = rows[bi][r, sl] + embs[bi][r, sl]

        def start_out(ci, bo):
            pltpu.async_copy(outs[bo], out_hbm.at[pl.ds(base + ci * c_rows, c_rows)], sos[bo])

        def wait_out(bo):
            pltpu.make_async_copy(outs[bo], out_hbm.at[pl.ds(base, c_rows)], sos[bo]).wait()

        def step(ci, bi, bo, prefetch, drain_out):
            if prefetch:
                start_in(ci + 2, (bi + 2) % NIN)
            wait_in(bi)
            if drain_out:
                wait_out(bo)
            add(bi, bo)
            start_out(ci, bo)

        start_in(0, 0)
        start_in(1, 1)
        step(0, 0, 0, True, False)
        step(1, 1, 1, True, False)

        @pl.loop(2, n_chunks - 2, step=NIN)
        def body(ci):
            for d in range(NIN):
                c = ci + d
                step(c, (2 + d) % NIN, d % NOUT, True, True)

        step(n_chunks - 2, (n_chunks - 2) % NIN, (n_chunks - 2) % NOUT, False, True)
        step(n_chunks - 1, (n_chunks - 1) % NIN, (n_chunks - 1) % NOUT, False, True)
        for bo in range(NOUT):
            wait_out(bo)

    return k


@jax.jit
def kernel(emb_vec, timesteps, pos_table):
    b, s, e = emb_vec.shape
    n = b * s
    emb2 = emb_vec.reshape(n, e)
    ts1 = timesteps.reshape(n)
    out = _make_kernel(n, 8)(emb2, ts1, pos_table)
    return out.reshape(b, s, e)
